# Initial kernel scaffold; baseline (speedup 1.0000x reference)
import jax
import jax.numpy as jnp
from jax.experimental import pallas as pl


def kernel(torch_ids, pads, table):
    raise NotImplementedError("write your pallas kernel here")


# speedup vs baseline: 1.8201x; 1.8201x over previous
import functools

import jax
import jax.numpy as jnp
from jax import lax
from jax.experimental import pallas as pl
from jax.experimental.pallas import tpu as pltpu
from jax.experimental.pallas import tpu_sc as plsc

NC = 2
NS = 16
NW = NC * NS

BATCH = 4096
MAX_LEN = 200
DIM = 64
B = BATCH * MAX_LEN
BPW = B // NW
C = 512
G = 128
SUB = C // G
NCH = BPW // C


def _body(ids_hbm, table_hbm, out_hbm, idx_v, rows0, rows1,
          gsem0, gsem1, osem0, osem1):
    wid = lax.axis_index("s") * NC + lax.axis_index("c")
    base = wid * BPW
    pltpu.sync_copy(ids_hbm.at[pl.ds(base, BPW)], idx_v)

    rows = (rows0, rows1)
    gsem = (gsem0, gsem1)
    osem = (osem0, osem1)

    def fire(c, b):
        for j in range(SUB):
            pltpu.async_copy(
                table_hbm.at[idx_v.at[pl.ds(c * C + j * G, G)]],
                rows[b].at[pl.ds(j * G, G)],
                gsem[b])

    def wait_gather(c, b):
        for j in range(SUB):
            pltpu.make_async_copy(
                table_hbm.at[idx_v.at[pl.ds(c * C + j * G, G)]],
                rows[b].at[pl.ds(j * G, G)],
                gsem[b]).wait()

    def write_out(c, b):
        pltpu.async_copy(rows[b], out_hbm.at[pl.ds(base + c * C, C)], osem[b])

    def wait_out(c, b):
        pltpu.make_async_copy(
            rows[b], out_hbm.at[pl.ds(base + c * C, C)], osem[b]).wait()

    fire(0, 0)
    wait_gather(0, 0)
    write_out(0, 0)
    fire(1, 1)

    def loop_body(t, carry):
        c1 = 2 * t - 1
        wait_gather(c1, 1)
        write_out(c1, 1)
        wait_out(c1 - 1, 0)
        fire(c1 + 1, 0)
        c2 = 2 * t
        wait_gather(c2, 0)
        write_out(c2, 0)
        wait_out(c2 - 1, 1)
        fire(c2 + 1, 1)
        return carry

    lax.fori_loop(1, NCH // 2, loop_body, 0, unroll=False)

    cl = NCH - 1
    wait_gather(cl, 1)
    write_out(cl, 1)
    wait_out(cl - 1, 0)
    wait_out(cl, 1)


@jax.jit
def _gather(ids_flat, table):
    mesh = plsc.VectorSubcoreMesh(
        core_axis_name="c", subcore_axis_name="s",
        num_cores=NC, num_subcores=NS)
    run = functools.partial(
        pl.kernel, mesh=mesh,
        compiler_params=pltpu.CompilerParams(use_tc_tiling_on_sc=False),
        out_type=jax.ShapeDtypeStruct((B, DIM), jnp.float32),
        scratch_types=[
            pltpu.VMEM((BPW,), jnp.int32),
            pltpu.VMEM((C, DIM), jnp.float32),
            pltpu.VMEM((C, DIM), jnp.float32),
            pltpu.SemaphoreType.DMA,
            pltpu.SemaphoreType.DMA,
            pltpu.SemaphoreType.DMA,
            pltpu.SemaphoreType.DMA,
        ])(_body)
    return run(ids_flat, table)


def kernel(torch_ids, pads, table):
    ids_flat = torch_ids.reshape(-1)
    out = _gather(ids_flat, table)
    return out.reshape(BATCH, MAX_LEN, DIM), pads
